# baseline (speedup 1.0000x reference)
import jax
import jax.numpy as jnp
from jax.experimental import pallas as pl
from jax.experimental.pallas import tpu as pltpu
from jax.experimental.pallas import tpu_sc as plsc

BATCH = 16384
NF = 64


_NC = 2
_NS = 16
_NW = _NC * _NS


_QW = 4 * NF


def _sc_gather_pair(u_qoff, i_qoff, u_tab1, i_tab1):
    mesh = plsc.VectorSubcoreMesh(core_axis_name="c", subcore_axis_name="s")
    n = u_qoff.shape[0]
    b_per_w = n // _NW
    out_type = (
        jax.ShapeDtypeStruct((n * _QW,), jnp.bfloat16),
        jax.ShapeDtypeStruct((n * _QW,), jnp.bfloat16),
    )

    @pl.kernel(
        out_type=out_type,
        mesh=mesh,
        scratch_types=[
            pltpu.VMEM((b_per_w,), jnp.int32),
            pltpu.VMEM((b_per_w,), jnp.int32),
            pltpu.VMEM((b_per_w * _QW,), jnp.bfloat16),
            pltpu.SemaphoreType.DMA,
        ],
    )
    def gather_kernel(u_idx_hbm, i_idx_hbm, u_tab_hbm, i_tab_hbm,
                      u_out_hbm, i_out_hbm, uidx_s, iidx_s, rows_v, sem):
        wid = jax.lax.axis_index("s") * _NC + jax.lax.axis_index("c")
        base = wid * b_per_w
        pltpu.sync_copy(u_idx_hbm.at[pl.ds(base, b_per_w)], uidx_s)
        pltpu.sync_copy(i_idx_hbm.at[pl.ds(base, b_per_w)], iidx_s)

        def one_table(idx_ref, tab_hbm, out_hbm):
            @pl.loop(0, b_per_w, step=16)
            def _issue(k):
                v = idx_ref[pl.ds(k, 16)]
                for j in range(16):
                    off = pl.multiple_of(v[j], _QW)
                    pltpu.async_copy(tab_hbm.at[pl.ds(off, _QW)],
                                     rows_v.at[pl.ds((k + j) * _QW, _QW)],
                                     sem)

            @pl.loop(0, b_per_w, step=128)
            def _drain(k):
                pltpu.make_async_copy(tab_hbm.at[pl.ds(0, 128 * _QW)],
                                      rows_v.at[pl.ds(k * _QW, 128 * _QW)],
                                      sem).wait()

            pltpu.sync_copy(rows_v, out_hbm.at[pl.ds(base * _QW,
                                                     b_per_w * _QW)])

        one_table(uidx_s, u_tab_hbm, u_out_hbm)
        one_table(iidx_s, i_tab_hbm, i_out_hbm)

    return gather_kernel(u_qoff, i_qoff, u_tab1, i_tab1)


_BT = 2048


def _mlp_body(u2_ref, i2_ref, usel_ref, isel_ref, w1u_ref, w1i_ref, b1_ref,
              w2_ref, b2_ref, w3_ref, b3_ref, out_ref):
    def pick(block_ref, sel2):
        x = block_ref[...]
        xa = x[:, :NF]
        xb = x[:, NF:]
        ce = jnp.where((sel2 & 1) == 1, xb, xa)
        co = jnp.concatenate([ce[1:], ce[:1]], axis=0)
        return jnp.where(sel2 >= 2, co, ce)

    u = pick(u2_ref, usel_ref[...])
    i = pick(i2_ref, isel_ref[...])
    h = jnp.dot(u, w1u_ref[...], preferred_element_type=jnp.float32)
    h += jnp.dot(i, w1i_ref[...], preferred_element_type=jnp.float32)
    h = jnp.maximum(h + b1_ref[...], 0.0).astype(jnp.bfloat16)
    h = jnp.dot(h, w2_ref[...], preferred_element_type=jnp.float32)
    h = jnp.maximum(h + b2_ref[...], 0.0).astype(jnp.bfloat16)
    out_ref[...] = (
        jnp.dot(h, w3_ref[...], preferred_element_type=jnp.float32)
        + b3_ref[...]
    )


def _tc_mlp(u2, i2, u_sel2, i_sel2, W1, b1, W2, b2, W3, b3):
    n2 = u_sel2.shape[0]
    w1u = W1[:NF].astype(jnp.bfloat16)
    w1i = W1[NF:].astype(jnp.bfloat16)
    grid = (n2 // (2 * _BT),)
    full = lambda *shape: pl.BlockSpec(shape, lambda g: (0,) * len(shape))
    out = pl.pallas_call(
        _mlp_body,
        grid=grid,
        in_specs=[
            pl.BlockSpec((2 * _BT, 2 * NF), lambda g: (g, 0)),
            pl.BlockSpec((2 * _BT, 2 * NF), lambda g: (g, 0)),
            pl.BlockSpec((2 * _BT, 1), lambda g: (g, 0)),
            pl.BlockSpec((2 * _BT, 1), lambda g: (g, 0)),
            full(NF, W1.shape[1]),
            full(NF, W1.shape[1]),
            full(1, b1.shape[0]),
            full(W2.shape[0], W2.shape[1]),
            full(1, b2.shape[0]),
            full(W3.shape[0], W3.shape[1]),
            full(1, 1),
        ],
        out_specs=pl.BlockSpec((2 * _BT, 1), lambda g: (g, 0)),
        out_shape=jax.ShapeDtypeStruct((n2, 1), jnp.float32),
    )(u2, i2, u_sel2, i_sel2, w1u, w1i, b1.reshape(1, -1),
      W2.astype(jnp.bfloat16), b2.reshape(1, -1), W3.astype(jnp.bfloat16),
      b3.reshape(1, 1))
    return out.reshape(n2 // 2, 2)[:, 0]


def kernel(users, items, user_table, item_table, W1, b1, W2, b2, W3, b3):
    users = users.astype(jnp.int32)
    items = items.astype(jnp.int32)
    u_tab1 = user_table.astype(jnp.bfloat16).reshape(-1)
    i_tab1 = item_table.astype(jnp.bfloat16).reshape(-1)
    u_qoff = (users >> 2) << 8
    i_qoff = (items >> 2) << 8
    u_flat, i_flat = _sc_gather_pair(u_qoff, i_qoff, u_tab1, i_tab1)
    n = users.shape[0]
    u2 = u_flat.reshape(2 * n, 2 * NF)
    i2 = i_flat.reshape(2 * n, 2 * NF)
    u_sel2 = jnp.repeat(users & 3, 2).reshape(-1, 1)
    i_sel2 = jnp.repeat(items & 3, 2).reshape(-1, 1)
    return _tc_mlp(u2, i2, u_sel2, i_sel2, W1, b1, W2, b2, W3, b3)


# speedup vs baseline: 1.6111x; 1.6111x over previous
import jax
import jax.numpy as jnp
from jax.experimental import pallas as pl
from jax.experimental.pallas import tpu as pltpu
from jax.experimental.pallas import tpu_sc as plsc

BATCH = 16384
NF = 64


_NC = 2
_NS = 16
_NW = _NC * _NS


def _sc_gather_pair(users, items, user_table, item_table):
    mesh = plsc.VectorSubcoreMesh(core_axis_name="c", subcore_axis_name="s")
    n = users.shape[0]
    b_per_w = n // _NW
    out_type = (
        jax.ShapeDtypeStruct((n, NF), jnp.float32),
        jax.ShapeDtypeStruct((n, NF), jnp.float32),
    )

    @pl.kernel(
        out_type=out_type,
        mesh=mesh,
        scratch_types=[
            pltpu.VMEM((b_per_w,), jnp.int32),
            pltpu.VMEM((b_per_w,), jnp.int32),
            pltpu.VMEM((b_per_w, NF), jnp.float32),
            pltpu.SemaphoreType.DMA,
        ],
    )
    def gather_kernel(u_idx_hbm, i_idx_hbm, u_tab_hbm, i_tab_hbm,
                      u_out_hbm, i_out_hbm, uidx_s, iidx_s, rows_v, sem):
        wid = jax.lax.axis_index("s") * _NC + jax.lax.axis_index("c")
        base = wid * b_per_w
        pltpu.sync_copy(u_idx_hbm.at[pl.ds(base, b_per_w)], uidx_s)
        pltpu.sync_copy(i_idx_hbm.at[pl.ds(base, b_per_w)], iidx_s)

        def one_table(idx_ref, tab_hbm, out_hbm):
            @pl.loop(0, b_per_w, step=16)
            def _issue(k):
                v = idx_ref[pl.ds(k, 16)]
                for j in range(16):
                    pltpu.async_copy(tab_hbm.at[pl.ds(v[j], 1)],
                                     rows_v.at[pl.ds(k + j, 1)], sem)

            @pl.loop(0, b_per_w, step=128)
            def _drain(k):
                pltpu.make_async_copy(tab_hbm.at[pl.ds(0, 128)],
                                      rows_v.at[pl.ds(k, 128)], sem).wait()

            pltpu.sync_copy(rows_v, out_hbm.at[pl.ds(base, b_per_w)])

        one_table(uidx_s, u_tab_hbm, u_out_hbm)
        one_table(iidx_s, i_tab_hbm, i_out_hbm)

    return gather_kernel(users, items, user_table, item_table)


_BT = 2048


def _mlp_body(u_ref, i_ref, w1u_ref, w1i_ref, b1_ref, w2_ref, b2_ref,
              w3_ref, b3_ref, out_ref):
    h = jnp.dot(u_ref[...], w1u_ref[...], preferred_element_type=jnp.float32)
    h += jnp.dot(i_ref[...], w1i_ref[...], preferred_element_type=jnp.float32)
    h = jnp.maximum(h + b1_ref[...], 0.0)
    h = jnp.dot(h, w2_ref[...], preferred_element_type=jnp.float32)
    h = jnp.maximum(h + b2_ref[...], 0.0)
    out_ref[...] = (
        jnp.dot(h, w3_ref[...], preferred_element_type=jnp.float32)
        + b3_ref[...]
    )


def _tc_mlp(u, i, W1, b1, W2, b2, W3, b3):
    n = u.shape[0]
    w1u = W1[:NF]
    w1i = W1[NF:]
    grid = (n // _BT,)
    full = lambda *shape: pl.BlockSpec(shape, lambda g: (0,) * len(shape))
    out = pl.pallas_call(
        _mlp_body,
        grid=grid,
        in_specs=[
            pl.BlockSpec((_BT, NF), lambda g: (g, 0)),
            pl.BlockSpec((_BT, NF), lambda g: (g, 0)),
            full(NF, W1.shape[1]),
            full(NF, W1.shape[1]),
            full(1, b1.shape[0]),
            full(W2.shape[0], W2.shape[1]),
            full(1, b2.shape[0]),
            full(W3.shape[0], W3.shape[1]),
            full(1, 1),
        ],
        out_specs=pl.BlockSpec((_BT, 1), lambda g: (g, 0)),
        out_shape=jax.ShapeDtypeStruct((n, 1), jnp.float32),
    )(u, i, w1u, w1i, b1.reshape(1, -1), W2, b2.reshape(1, -1), W3,
      b3.reshape(1, 1))
    return out.reshape(n)


def kernel(users, items, user_table, item_table, W1, b1, W2, b2, W3, b3):
    users = users.astype(jnp.int32)
    items = items.astype(jnp.int32)
    u, i = _sc_gather_pair(users, items, user_table, item_table)
    return _tc_mlp(u, i, W1, b1, W2, b2, W3, b3)
